# baseline (speedup 1.0000x reference)
import functools

import jax
import jax.numpy as jnp
from jax import lax
from jax.experimental import pallas as pl
from jax.experimental.pallas import tpu as pltpu
from jax.experimental.pallas import tpu_sc as plsc

_N = 16384
_S = 4096
_DP = 128


def _fps_body(px_ref, py_ref, pz_ref, pxs_ref, pys_ref, pzs_ref,
              idx_ref, kx_ref, ky_ref, kz_ref, dists_ref):
    lin = (lax.broadcasted_iota(jnp.int32, (128, 128), 0) * 128
           + lax.broadcasted_iota(jnp.int32, (128, 128), 1))
    lin32 = (lax.broadcasted_iota(jnp.int32, (32, 128), 0) * 128
             + lax.broadcasted_iota(jnp.int32, (32, 128), 1))

    lin_f = lin.astype(jnp.float32)
    lin3f = lin_f.reshape(16, 8, 128)
    bigf = jnp.float32(2.0 ** 30)

    qx0 = pxs_ref[0]
    qy0 = pys_ref[0]
    qz0 = pzs_ref[0]
    dx = px_ref[...] - qx0
    dy = py_ref[...] - qy0
    dz = pz_ref[...] - qz0
    d0 = (dx * dx + dz * dz) + dy * dy
    dists_ref[...] = d0
    idx_ref[...] = jnp.zeros((32, 128), jnp.int32)
    sel0 = lin32 == 0
    kx_ref[...] = jnp.where(sel0, qx0, 0.0)
    ky_ref[...] = jnp.where(sel0, qy0, 0.0)
    kz_ref[...] = jnp.where(sel0, qz0, 0.0)
    d03 = d0.reshape(16, 8, 128)
    mv0 = jnp.max(d03, axis=0)
    il0 = jnp.min(jnp.where(d03 == mv0[None], lin3f, bigf), axis=0)
    m0 = jnp.max(mv0)

    def body(i, carry):
        m, mv, il = carry
        nxt = jnp.min(jnp.where(mv == m, il, bigf)).astype(jnp.int32)
        qx = pxs_ref[nxt]
        qy = pys_ref[nxt]
        qz = pzs_ref[nxt]
        dists = dists_ref[...]
        ddx = px_ref[...] - qx
        ddy = py_ref[...] - qy
        ddz = pz_ref[...] - qz
        d = (ddx * ddx + ddz * ddz) + ddy * ddy
        new = jnp.minimum(dists, d)
        dists_ref[...] = new
        sel = lin32 == i
        idx_ref[...] = jnp.where(sel, nxt, idx_ref[...])
        kx_ref[...] = jnp.where(sel, qx, kx_ref[...])
        ky_ref[...] = jnp.where(sel, qy, ky_ref[...])
        kz_ref[...] = jnp.where(sel, qz, kz_ref[...])
        new3 = new.reshape(16, 8, 128)
        mv2 = jnp.max(new3, axis=0)
        m2 = jnp.max(mv2)
        il2 = jnp.min(jnp.where(new3 == mv2[None], lin3f, bigf), axis=0)
        return (m2, mv2, il2)

    lax.fori_loop(1, _S, body, (m0, mv0, il0))


_fps = pl.pallas_call(
    _fps_body,
    in_specs=[
        pl.BlockSpec((128, 128), lambda: (0, 0)),
        pl.BlockSpec((128, 128), lambda: (0, 0)),
        pl.BlockSpec((128, 128), lambda: (0, 0)),
        pl.BlockSpec(memory_space=pltpu.SMEM),
        pl.BlockSpec(memory_space=pltpu.SMEM),
        pl.BlockSpec(memory_space=pltpu.SMEM),
    ],
    out_shape=(
        jax.ShapeDtypeStruct((32, 128), jnp.int32),
        jax.ShapeDtypeStruct((32, 128), jnp.float32),
        jax.ShapeDtypeStruct((32, 128), jnp.float32),
        jax.ShapeDtypeStruct((32, 128), jnp.float32),
    ),
    scratch_shapes=[pltpu.VMEM((128, 128), jnp.float32)],
)


_QB = 256


def _knn_body(qf_ref, kt_ref, sqk_ref, idxf_ref, g_ref):
    qf = qf_ref[...]
    qb = qf.astype(jnp.bfloat16)
    kt = kt_ref[...]
    qk = lax.dot_general(qb, kt, (((1,), (0,)), ((), ())),
                         preferred_element_type=jnp.float32)
    x = qf[:, 0:1]
    y = qf[:, 1:2]
    z = qf[:, 2:3]
    sqq = (x * x + z * z) + y * y
    d2 = (sqq - 2.0 * qk) + sqk_ref[...]
    m = jnp.min(d2, axis=1, keepdims=True)
    lane = lax.broadcasted_iota(jnp.int32, (_QB, _S), 1)
    nn = jnp.min(jnp.where(d2 == m, lane, jnp.int32(2 ** 30)),
                 axis=1, keepdims=True)
    gf = jnp.min(jnp.where(lane == nn, idxf_ref[...], jnp.float32(2.0 ** 30)),
                 axis=1, keepdims=True)
    g_ref[...] = gf.astype(jnp.int32)


_knn = pl.pallas_call(
    _knn_body,
    grid=(_N // _QB,),
    in_specs=[
        pl.BlockSpec((_QB, 16), lambda b: (b, 0)),
        pl.BlockSpec((16, _S), lambda b: (0, 0)),
        pl.BlockSpec((1, _S), lambda b: (0, 0)),
        pl.BlockSpec((1, _S), lambda b: (0, 0)),
    ],
    out_specs=pl.BlockSpec((_QB, 1), lambda b: (b, 0)),
    out_shape=jax.ShapeDtypeStruct((_N, 1), jnp.int32),
)


_NC = 2
_NW = _NC * 16
_QPW = _N // _NW


def _gather_rows_body(table_ref, ind_ref, out_ref, ind_v, rows_v, sem):
    wid = lax.axis_index("s") * _NC + lax.axis_index("c")
    bpw = ind_v.shape[0]
    base = wid * bpw
    pltpu.sync_copy(ind_ref.at[pl.ds(base, bpw)], ind_v)
    pltpu.async_copy(table_ref.at[ind_v], rows_v, sem).wait()
    pltpu.sync_copy(rows_v, out_ref.at[pl.ds(base, bpw)])


@functools.lru_cache(maxsize=None)
def _make_gather(n_rows, n_idx):
    bpw = n_idx // _NW
    return pl.kernel(
        _gather_rows_body,
        out_type=jax.ShapeDtypeStruct((n_idx, _DP), jnp.float32),
        mesh=plsc.VectorSubcoreMesh(core_axis_name="c", subcore_axis_name="s",
                                    num_cores=_NC, num_subcores=_NW // _NC),
        scratch_types=[
            pltpu.VMEM((bpw,), jnp.int32),
            pltpu.VMEM((bpw, _DP), jnp.float32),
            pltpu.SemaphoreType.DMA,
        ],
    )


def kernel(x, pos, batch):
    posT = pos.T
    px = posT[0].reshape(128, 128)
    py = posT[1].reshape(128, 128)
    pz = posT[2].reshape(128, 128)

    idx32, kx32, ky32, kz32 = _fps(px, py, pz, posT[0], posT[1], posT[2])

    kx = kx32.reshape(_S)
    ky = ky32.reshape(_S)
    kz = kz32.reshape(_S)
    sqk = ((kx * kx + kz * kz) + ky * ky)[None, :]
    kt = jnp.zeros((16, _S), jnp.bfloat16)
    kt = kt.at[0].set(kx.astype(jnp.bfloat16))
    kt = kt.at[1].set(ky.astype(jnp.bfloat16))
    kt = kt.at[2].set(kz.astype(jnp.bfloat16))
    qf = jnp.pad(pos, ((0, 0), (0, 13)))
    idxf = idx32.reshape(1, _S).astype(jnp.float32)

    g2 = _knn(qf, kt, sqk, idxf)

    table = jnp.concatenate(
        [x, pos, jnp.zeros((_N, _DP - 67), x.dtype)], axis=1)
    interp = _make_gather(_N, _N)(table, g2.reshape(_N))

    out_x = jnp.concatenate([x, pos, interp[:, :67]], axis=1)
    return (out_x, jnp.zeros((_N, 3), pos.dtype), batch)


# speedup vs baseline: 1.0678x; 1.0678x over previous
import functools

import jax
import jax.numpy as jnp
from jax import lax
from jax.experimental import pallas as pl
from jax.experimental.pallas import tpu as pltpu
from jax.experimental.pallas import tpu_sc as plsc

_N = 16384
_S = 4096
_DP = 128


def _fps_body(px_ref, py_ref, pz_ref, pxs_ref, pys_ref, pzs_ref,
              idx_ref, dists_ref):
    lin = (lax.broadcasted_iota(jnp.int32, (128, 128), 0) * 128
           + lax.broadcasted_iota(jnp.int32, (128, 128), 1))
    lin32 = (lax.broadcasted_iota(jnp.int32, (32, 128), 0) * 128
             + lax.broadcasted_iota(jnp.int32, (32, 128), 1))

    lin_f = lin.astype(jnp.float32)
    bigf = jnp.float32(2.0 ** 30)

    qx0 = pxs_ref[0]
    qy0 = pys_ref[0]
    qz0 = pzs_ref[0]
    dx = px_ref[...] - qx0
    dy = py_ref[...] - qy0
    dz = pz_ref[...] - qz0
    d0 = (dx * dx + dz * dz) + dy * dy
    dists_ref[...] = d0
    idx_ref[...] = jnp.zeros((32, 128), jnp.int32)
    mv0 = jnp.max(d0, axis=0)
    il0 = jnp.min(jnp.where(d0 == mv0[None, :], lin_f, bigf), axis=0)
    m0 = jnp.max(mv0)

    def body(i, carry):
        m, mv, il = carry
        nxt = jnp.min(jnp.where(mv == m, il, bigf)).astype(jnp.int32)
        qx = pxs_ref[nxt]
        qy = pys_ref[nxt]
        qz = pzs_ref[nxt]
        dists = dists_ref[...]
        ddx = px_ref[...] - qx
        ddy = py_ref[...] - qy
        ddz = pz_ref[...] - qz
        d = (ddx * ddx + ddz * ddz) + ddy * ddy
        new = jnp.minimum(dists, d)
        dists_ref[...] = new
        idx_ref[...] = jnp.where(lin32 == i, nxt, idx_ref[...])
        mv2 = jnp.max(new, axis=0)
        il2 = jnp.min(jnp.where(new == mv2[None, :], lin_f, bigf), axis=0)
        m2 = jnp.max(mv2)
        return (m2, mv2, il2)

    lax.fori_loop(1, _S, body, (m0, mv0, il0), unroll=2)


_fps = pl.pallas_call(
    _fps_body,
    in_specs=[
        pl.BlockSpec((128, 128), lambda: (0, 0)),
        pl.BlockSpec((128, 128), lambda: (0, 0)),
        pl.BlockSpec((128, 128), lambda: (0, 0)),
        pl.BlockSpec(memory_space=pltpu.SMEM),
        pl.BlockSpec(memory_space=pltpu.SMEM),
        pl.BlockSpec(memory_space=pltpu.SMEM),
    ],
    out_shape=jax.ShapeDtypeStruct((32, 128), jnp.int32),
    scratch_shapes=[pltpu.VMEM((128, 128), jnp.float32)],
)


_QB = 256


def _knn_body(qf_ref, kt_ref, sqk_ref, nn_ref):
    qf = qf_ref[...]
    qb = qf.astype(jnp.bfloat16)
    kt = kt_ref[...]
    qk = lax.dot_general(qb, kt, (((1,), (0,)), ((), ())),
                         preferred_element_type=jnp.float32)
    x = qf[:, 0:1]
    y = qf[:, 1:2]
    z = qf[:, 2:3]
    sqq = (x * x + z * z) + y * y
    d2 = (sqq - 2.0 * qk) + sqk_ref[...]
    m = jnp.min(d2, axis=1, keepdims=True)
    lane = lax.broadcasted_iota(jnp.int32, (_QB, _S), 1)
    nn_ref[...] = jnp.min(jnp.where(d2 == m, lane, jnp.int32(2 ** 30)),
                          axis=1, keepdims=True)


_knn = pl.pallas_call(
    _knn_body,
    grid=(_N // _QB,),
    in_specs=[
        pl.BlockSpec((_QB, 16), lambda b: (b, 0)),
        pl.BlockSpec((16, _S), lambda b: (0, 0)),
        pl.BlockSpec((1, _S), lambda b: (0, 0)),
    ],
    out_specs=pl.BlockSpec((_QB, 1), lambda b: (b, 0)),
    out_shape=jax.ShapeDtypeStruct((_N, 1), jnp.int32),
)


_NC = 2
_NW = _NC * 16
_QPW = _N // _NW


def _gather_rows_body(table_ref, ind_ref, out_ref, ind_v, rows_v, sem):
    wid = lax.axis_index("s") * _NC + lax.axis_index("c")
    bpw = ind_v.shape[0]
    base = wid * bpw
    pltpu.sync_copy(ind_ref.at[pl.ds(base, bpw)], ind_v)
    pltpu.async_copy(table_ref.at[ind_v], rows_v, sem).wait()
    pltpu.sync_copy(rows_v, out_ref.at[pl.ds(base, bpw)])


@functools.lru_cache(maxsize=None)
def _make_gather(n_idx):
    bpw = n_idx // _NW
    return pl.kernel(
        _gather_rows_body,
        out_type=jax.ShapeDtypeStruct((n_idx, _DP), jnp.float32),
        mesh=plsc.VectorSubcoreMesh(core_axis_name="c", subcore_axis_name="s",
                                    num_cores=_NC, num_subcores=_NW // _NC),
        scratch_types=[
            pltpu.VMEM((bpw,), jnp.int32),
            pltpu.VMEM((bpw, _DP), jnp.float32),
            pltpu.SemaphoreType.DMA,
        ],
    )


def kernel(x, pos, batch):
    posT = pos.T
    px = posT[0].reshape(128, 128)
    py = posT[1].reshape(128, 128)
    pz = posT[2].reshape(128, 128)

    idx32 = _fps(px, py, pz, posT[0], posT[1], posT[2])
    idx = idx32.reshape(_S)

    table = jnp.concatenate(
        [x, pos, jnp.zeros((_N, _DP - 67), x.dtype)], axis=1)
    lr = _make_gather(_S)(table, idx)

    kx = lr[:, 64]
    ky = lr[:, 65]
    kz = lr[:, 66]
    sqk = ((kx * kx + kz * kz) + ky * ky)[None, :]
    kt = jnp.zeros((16, _S), jnp.bfloat16)
    kt = kt.at[0].set(kx.astype(jnp.bfloat16))
    kt = kt.at[1].set(ky.astype(jnp.bfloat16))
    kt = kt.at[2].set(kz.astype(jnp.bfloat16))
    qf = jnp.pad(pos, ((0, 0), (0, 13)))

    nn2 = _knn(qf, kt, sqk)
    nn = nn2.reshape(_N)
    interp = _make_gather(_N)(lr, nn)

    out_x = jnp.concatenate([x, pos, interp[:, :67]], axis=1)
    return (out_x, jnp.zeros((_N, 3), pos.dtype), batch)


# speedup vs baseline: 1.0767x; 1.0083x over previous
import functools

import jax
import jax.numpy as jnp
from jax import lax
from jax.experimental import pallas as pl
from jax.experimental.pallas import tpu as pltpu
from jax.experimental.pallas import tpu_sc as plsc

_N = 16384
_S = 4096
_DP = 128


def _fps_body(px_ref, py_ref, pz_ref, pxs_ref, pys_ref, pzs_ref,
              idx_ref, dists_ref):
    lin = (lax.broadcasted_iota(jnp.int32, (128, 128), 0) * 128
           + lax.broadcasted_iota(jnp.int32, (128, 128), 1))
    lin32 = (lax.broadcasted_iota(jnp.int32, (32, 128), 0) * 128
             + lax.broadcasted_iota(jnp.int32, (32, 128), 1))

    lin_f = lin.astype(jnp.float32)
    bigf = jnp.float32(2.0 ** 30)

    qx0 = pxs_ref[0]
    qy0 = pys_ref[0]
    qz0 = pzs_ref[0]
    dx = px_ref[...] - qx0
    dy = py_ref[...] - qy0
    dz = pz_ref[...] - qz0
    d0 = (dx * dx + dz * dz) + dy * dy
    dists_ref[...] = d0
    idx_ref[...] = jnp.zeros((32, 128), jnp.int32)
    mv0 = jnp.max(d0, axis=0)
    il0 = jnp.min(jnp.where(d0 == mv0[None, :], lin_f, bigf), axis=0)
    m0 = jnp.max(mv0)

    def body(i, carry):
        m, mv, il = carry
        nxt = jnp.min(jnp.where(mv == m, il, bigf)).astype(jnp.int32)
        qx = pxs_ref[nxt]
        qy = pys_ref[nxt]
        qz = pzs_ref[nxt]
        dists = dists_ref[...]
        ddx = px_ref[...] - qx
        ddy = py_ref[...] - qy
        ddz = pz_ref[...] - qz
        d = (ddx * ddx + ddz * ddz) + ddy * ddy
        new = jnp.minimum(dists, d)
        dists_ref[...] = new
        idx_ref[...] = jnp.where(lin32 == i, nxt, idx_ref[...])
        mv2 = jnp.max(new, axis=0)
        il2 = jnp.min(jnp.where(new == mv2[None, :], lin_f, bigf), axis=0)
        m2 = jnp.max(mv2)
        return (m2, mv2, il2)

    lax.fori_loop(1, _S, body, (m0, mv0, il0), unroll=2)


_fps = pl.pallas_call(
    _fps_body,
    in_specs=[
        pl.BlockSpec((128, 128), lambda: (0, 0)),
        pl.BlockSpec((128, 128), lambda: (0, 0)),
        pl.BlockSpec((128, 128), lambda: (0, 0)),
        pl.BlockSpec(memory_space=pltpu.SMEM),
        pl.BlockSpec(memory_space=pltpu.SMEM),
        pl.BlockSpec(memory_space=pltpu.SMEM),
    ],
    out_shape=jax.ShapeDtypeStruct((32, 128), jnp.int32),
    scratch_shapes=[pltpu.VMEM((128, 128), jnp.float32)],
)


_QB = 512


def _knn_body(qf_ref, kt_ref, sqk_ref, nn_ref):
    qf = qf_ref[...]
    qb = qf.astype(jnp.bfloat16)
    kt = kt_ref[...]
    qk = lax.dot_general(qb, kt, (((1,), (0,)), ((), ())),
                         preferred_element_type=jnp.float32)
    x = qf[:, 0:1]
    y = qf[:, 1:2]
    z = qf[:, 2:3]
    sqq = (x * x + z * z) + y * y
    d2 = (sqq - 2.0 * qk) + sqk_ref[...]
    m = jnp.min(d2, axis=1, keepdims=True)
    lane = lax.broadcasted_iota(jnp.int32, (_QB, _S), 1)
    nn_ref[...] = jnp.min(jnp.where(d2 == m, lane, jnp.int32(2 ** 30)),
                          axis=1, keepdims=True)


_knn = pl.pallas_call(
    _knn_body,
    grid=(_N // _QB,),
    in_specs=[
        pl.BlockSpec((_QB, 16), lambda b: (b, 0)),
        pl.BlockSpec((16, _S), lambda b: (0, 0)),
        pl.BlockSpec((1, _S), lambda b: (0, 0)),
    ],
    out_specs=pl.BlockSpec((_QB, 1), lambda b: (b, 0)),
    out_shape=jax.ShapeDtypeStruct((_N, 1), jnp.int32),
)


_NC = 2
_NW = _NC * 16
_QPW = _N // _NW


def _gather_rows_body(table_ref, ind_ref, out_ref, ind_v, rows_v, sem):
    wid = lax.axis_index("s") * _NC + lax.axis_index("c")
    bpw = ind_v.shape[0]
    base = wid * bpw
    pltpu.sync_copy(ind_ref.at[pl.ds(base, bpw)], ind_v)
    pltpu.async_copy(table_ref.at[ind_v], rows_v, sem).wait()
    pltpu.sync_copy(rows_v, out_ref.at[pl.ds(base, bpw)])


@functools.lru_cache(maxsize=None)
def _make_gather(n_idx):
    bpw = n_idx // _NW
    return pl.kernel(
        _gather_rows_body,
        out_type=jax.ShapeDtypeStruct((n_idx, _DP), jnp.float32),
        mesh=plsc.VectorSubcoreMesh(core_axis_name="c", subcore_axis_name="s",
                                    num_cores=_NC, num_subcores=_NW // _NC),
        scratch_types=[
            pltpu.VMEM((bpw,), jnp.int32),
            pltpu.VMEM((bpw, _DP), jnp.float32),
            pltpu.SemaphoreType.DMA,
        ],
    )


def kernel(x, pos, batch):
    posT = pos.T
    px = posT[0].reshape(128, 128)
    py = posT[1].reshape(128, 128)
    pz = posT[2].reshape(128, 128)

    idx32 = _fps(px, py, pz, posT[0], posT[1], posT[2])
    idx = idx32.reshape(_S)

    table = jnp.concatenate(
        [x, pos, jnp.zeros((_N, _DP - 67), x.dtype)], axis=1)
    lr = _make_gather(_S)(table, idx)

    kx = lr[:, 64]
    ky = lr[:, 65]
    kz = lr[:, 66]
    sqk = ((kx * kx + kz * kz) + ky * ky)[None, :]
    kt = jnp.zeros((16, _S), jnp.bfloat16)
    kt = kt.at[0].set(kx.astype(jnp.bfloat16))
    kt = kt.at[1].set(ky.astype(jnp.bfloat16))
    kt = kt.at[2].set(kz.astype(jnp.bfloat16))
    qf = jnp.pad(pos, ((0, 0), (0, 13)))

    nn2 = _knn(qf, kt, sqk)
    nn = nn2.reshape(_N)
    interp = _make_gather(_N)(lr, nn)

    out_x = jnp.concatenate([x, pos, interp[:, :67]], axis=1)
    return (out_x, jnp.zeros((_N, 3), pos.dtype), batch)
